# Initial kernel scaffold; baseline (speedup 1.0000x reference)
import jax
import jax.numpy as jnp
from jax.experimental import pallas as pl


def kernel(input, pos_table):
    raise NotImplementedError("write your pallas kernel here")


# speedup vs baseline: 1.6738x; 1.6738x over previous
import jax
import jax.numpy as jnp
from jax.experimental import pallas as pl


def _add_body(in_ref, pos_ref, out_ref):
    out_ref[...] = in_ref[...] + pos_ref[...][:, None, :]


def kernel(input, pos_table):
    S, B, E = input.shape
    BS = 256
    grid = (S // BS,)
    return pl.pallas_call(
        _add_body,
        grid=grid,
        in_specs=[
            pl.BlockSpec((BS, B, E), lambda i: (i, 0, 0)),
            pl.BlockSpec((BS, E), lambda i: (i, 0)),
        ],
        out_specs=pl.BlockSpec((BS, B, E), lambda i: (i, 0, 0)),
        out_shape=jax.ShapeDtypeStruct((S, B, E), input.dtype),
    )(input, pos_table[:S])
